# Initial kernel scaffold; baseline (speedup 1.0000x reference)
import jax
import jax.numpy as jnp
from jax.experimental import pallas as pl


def kernel(x, edge_index, W, b):
    raise NotImplementedError("write your pallas kernel here")


# speedup vs baseline: 6.2109x; 6.2109x over previous
import functools

import jax
import jax.numpy as jnp
from jax import lax
from jax.experimental import pallas as pl
from jax.experimental.pallas import tpu as pltpu
from jax.experimental.pallas import tpu_sc as plsc

N_NODES = 10000
N_EDGES = 160000
D_IN = 256
D_OUT = 256
HALF = D_OUT // 2

RB = 1000

N_SUB = 16
E_PER_SUB = N_EDGES // N_SUB
CH = 128
N_FULL = E_PER_SUB // CH
TAIL = E_PER_SUB - N_FULL * CH
ROWS_PER_SUB = 624
ROWS_TAIL = N_NODES - N_SUB * ROWS_PER_SUB


def _matmul_body(x_ref, wt_ref, b_ref, h2_ref):
    h2_ref[...] = (
        jnp.dot(x_ref[...], wt_ref[...], preferred_element_type=jnp.float32)
        + b_ref[...]
    )


def _matmul_halves(x, wt, b2):
    grid = (N_NODES // RB, 2)
    return pl.pallas_call(
        _matmul_body,
        grid=grid,
        in_specs=[
            pl.BlockSpec((RB, D_IN), lambda i, j: (i, 0)),
            pl.BlockSpec((D_IN, HALF), lambda i, j: (0, j)),
            pl.BlockSpec((1, HALF), lambda i, j: (0, j)),
        ],
        out_specs=pl.BlockSpec(
            (RB, HALF), lambda i, j: (j * (N_NODES // RB) + i, 0)
        ),
        out_shape=jax.ShapeDtypeStruct((2 * N_NODES, HALF), jnp.float32),
    )(x, wt, b2)


def _scatter_add(h2, src, dst):
    mesh = plsc.VectorSubcoreMesh(core_axis_name="c", subcore_axis_name="s")

    @functools.partial(
        pl.kernel,
        out_type=jax.ShapeDtypeStruct((2 * N_NODES, HALF), jnp.float32),
        mesh=mesh,
        scratch_types=[
            pltpu.VMEM((CH,), jnp.int32),
            pltpu.VMEM((CH,), jnp.int32),
            pltpu.VMEM((CH, HALF), jnp.float32),
            pltpu.VMEM((TAIL,), jnp.int32),
            pltpu.VMEM((TAIL,), jnp.int32),
            pltpu.VMEM((TAIL, HALF), jnp.float32),
            pltpu.VMEM_SHARED((N_NODES, HALF), jnp.float32),
            pltpu.SemaphoreType.DMA,
        ],
    )
    def k(h2_hbm, src_hbm, dst_hbm, out_hbm,
          src_v, dst_v, rows_v, src_t, dst_t, rows_t, acc, sem):
        cid = lax.axis_index("c")
        sid = lax.axis_index("s")
        row0 = cid * N_NODES

        init_lo = sid * ROWS_PER_SUB
        pltpu.sync_copy(
            h2_hbm.at[pl.ds(row0 + init_lo, ROWS_PER_SUB)],
            acc.at[pl.ds(init_lo, ROWS_PER_SUB)],
        )

        @pl.when(sid == 0)
        def _():
            lo = N_SUB * ROWS_PER_SUB
            pltpu.sync_copy(
                h2_hbm.at[pl.ds(row0 + lo, ROWS_TAIL)],
                acc.at[pl.ds(lo, ROWS_TAIL)],
            )

        plsc.subcore_barrier()

        ebase = sid * E_PER_SUB

        def add_base(idx_ref, n):
            for i in range(n // 16):
                sl = pl.ds(i * 16, 16)
                idx_ref[sl] = idx_ref[sl] + row0

        def do_chunk(off, n, src_r, dst_r, rows_r):
            pltpu.sync_copy(src_hbm.at[pl.ds(off, n)], src_r)
            pltpu.sync_copy(dst_hbm.at[pl.ds(off, n)], dst_r)
            add_base(src_r, n)
            pltpu.async_copy(h2_hbm.at[src_r], rows_r, sem).wait()
            pltpu.sync_copy(rows_r, acc.at[dst_r], add=True)

        def body(kk, carry):
            do_chunk(ebase + kk * CH, CH, src_v, dst_v, rows_v)
            return carry

        lax.fori_loop(0, N_FULL, body, 0)
        do_chunk(ebase + N_FULL * CH, TAIL, src_t, dst_t, rows_t)

        plsc.subcore_barrier()
        pltpu.sync_copy(
            acc.at[pl.ds(init_lo, ROWS_PER_SUB)],
            out_hbm.at[pl.ds(row0 + init_lo, ROWS_PER_SUB)],
        )

        @pl.when(sid == 0)
        def _():
            lo = N_SUB * ROWS_PER_SUB
            pltpu.sync_copy(
                acc.at[pl.ds(lo, ROWS_TAIL)],
                out_hbm.at[pl.ds(row0 + lo, ROWS_TAIL)],
            )

    return k(h2, src, dst)


def kernel(x, edge_index, W, b):
    src = edge_index[0].astype(jnp.int32)
    dst = edge_index[1].astype(jnp.int32)
    h2 = _matmul_halves(x, W.T, b.reshape(1, -1))
    out2 = _scatter_add(h2, src, dst)
    return jnp.concatenate([out2[:N_NODES], out2[N_NODES:]], axis=1)
